# Initial kernel scaffold; baseline (speedup 1.0000x reference)
import jax
import jax.numpy as jnp
from jax.experimental import pallas as pl


def kernel(x, edge_index, batch, W1, b1, W2, b2, W3, b3):
    raise NotImplementedError("write your pallas kernel here")


# speedup vs baseline: 27.4465x; 27.4465x over previous
import functools

import jax
import jax.numpy as jnp
from jax import lax
from jax.experimental import pallas as pl
from jax.experimental.pallas import tpu as pltpu
from jax.experimental.pallas import tpu_sc as plsc

N = 100000
E = 3200000
G = 64
F1 = 16

NC, NS = 2, 16
NW = NC * NS
CH = 128
SJ = 16
BLK = 2048
NSUB = BLK // CH

N_PAD = 100352
NR = N_PAD // CH
NCHUNK = N_PAD // BLK
RD = N_PAD // NS
ZR = RD // 8

ER = 25088
E_PAD = ER * CH
TR = ER // NW
OI = TR // SJ


def _edge_mesh():
    return plsc.VectorSubcoreMesh(core_axis_name="c", subcore_axis_name="s")


def _make_edge_pass(F):
    @functools.partial(
        pl.kernel,
        out_type=jax.ShapeDtypeStruct((NC, N_PAD, F), jnp.float32),
        mesh=_edge_mesh(),
        scratch_types=[
            pltpu.VMEM((SJ, CH), jnp.int32),
            pltpu.VMEM((SJ, CH), jnp.int32),
            pltpu.VMEM((CH, F), jnp.float32),
            pltpu.VMEM((ZR, F), jnp.float32),
            pltpu.VMEM_SHARED((N_PAD, F), jnp.float32),
            pltpu.SemaphoreType.DMA,
        ],
        compiler_params=pltpu.CompilerParams(use_tc_tiling_on_sc=False),
    )
    def edge_pass(hs, rows, cols, zc, out, rowb, colb, rv, zb, agg, sem):
        c = lax.axis_index("c")
        s = lax.axis_index("s")
        wid = s * NC + c

        pltpu.sync_copy(zc, zb)
        for z in range(RD // ZR):
            pltpu.sync_copy(zb, agg.at[pl.ds(s * RD + z * ZR, ZR)])
        plsc.subcore_barrier()

        def outer(i, carry):
            base = wid * TR + i * SJ
            pltpu.sync_copy(rows.at[pl.ds(base, SJ)], rowb)
            pltpu.sync_copy(cols.at[pl.ds(base, SJ)], colb)
            for j in range(SJ):
                pltpu.async_copy(hs.at[rowb.at[j]], rv, sem).wait()
                pltpu.sync_copy(rv, agg.at[colb.at[j]], add=True)
            return carry

        lax.fori_loop(0, OI, outer, 0)
        plsc.subcore_barrier()

        for z in range(RD // ZR):
            sl = pl.ds(s * RD + z * ZR, ZR)
            pltpu.sync_copy(agg.at[sl], zb)
            pltpu.sync_copy(zb, out.at[c, sl])

    return edge_pass


_edge_pass_16 = _make_edge_pass(16)
_edge_pass_8 = _make_edge_pass(8)


@functools.partial(
    pl.kernel,
    out_type=jax.ShapeDtypeStruct((NC, N_PAD), jnp.float32),
    mesh=_edge_mesh(),
    scratch_types=[
        pltpu.VMEM((SJ, CH), jnp.int32),
        pltpu.VMEM((CH,), jnp.float32),
        pltpu.VMEM((RD,), jnp.float32),
        pltpu.VMEM_SHARED((N_PAD,), jnp.float32),
        pltpu.SemaphoreType.DMA,
    ],
    compiler_params=pltpu.CompilerParams(use_tc_tiling_on_sc=False),
)
def _deg_pass(cols, ones_h, zeros_h, out, colb, onesb, zb, deg, sem):
    c = lax.axis_index("c")
    s = lax.axis_index("s")
    wid = s * NC + c

    pltpu.sync_copy(ones_h, onesb)
    pltpu.sync_copy(zeros_h, zb)
    pltpu.sync_copy(zb, deg.at[pl.ds(s * RD, RD)])
    plsc.subcore_barrier()

    def outer(i, carry):
        base = wid * TR + i * SJ
        pltpu.sync_copy(cols.at[pl.ds(base, SJ)], colb)
        for j in range(SJ):
            pltpu.sync_copy(onesb, deg.at[colb.at[j]], add=True)
        return carry

    lax.fori_loop(0, OI, outer, 0)
    plsc.subcore_barrier()

    pltpu.sync_copy(deg.at[pl.ds(s * RD, RD)], zb)
    pltpu.sync_copy(zb, out.at[c, pl.ds(s * RD, RD)])


def _dinv_body(deg_ref, dinv_ref):
    deg = deg_ref[0] + deg_ref[1]
    dinv_ref[...] = jnp.where(deg > 0, lax.rsqrt(deg), 0.0)


def _dinv(deg2):
    return pl.pallas_call(
        _dinv_body,
        in_specs=[pl.BlockSpec((NC, NR, CH), lambda: (0, 0, 0))],
        out_specs=pl.BlockSpec((NR, CH), lambda: (0, 0)),
        out_shape=jax.ShapeDtypeStruct((NR, CH), jnp.float32),
    )(deg2)


def _prep1_body(x_ref, w1_ref, dv_ref, hs1_ref):
    h = jnp.dot(x_ref[...], w1_ref[...], preferred_element_type=jnp.float32)
    hs1_ref[...] = h * dv_ref[...]


def _prep1(xp, W1, dv16):
    return pl.pallas_call(
        _prep1_body,
        grid=(NCHUNK,),
        in_specs=[
            pl.BlockSpec((BLK, 8), lambda i: (i, 0)),
            pl.BlockSpec((8, F1), lambda i: (0, 0)),
            pl.BlockSpec((BLK, F1), lambda i: (i, 0)),
        ],
        out_specs=pl.BlockSpec((BLK, F1), lambda i: (i, 0)),
        out_shape=jax.ShapeDtypeStruct((N_PAD, F1), jnp.float32),
    )(xp, W1, dv16)


def _mid_body(agg1_ref, hs1_ref, dv16_ref, dv8_ref, b1_ref, w2a_ref, w2b_ref,
              hs2a_ref, hs2b_ref):
    dv = dv16_ref[...]
    a = agg1_ref[0] + agg1_ref[1] + hs1_ref[...]
    out1 = jnp.maximum(a * dv + b1_ref[...], 0.0)
    hs2a_ref[...] = jnp.dot(out1, w2a_ref[...],
                            preferred_element_type=jnp.float32) * dv
    hs2b_ref[...] = jnp.dot(out1, w2b_ref[...],
                            preferred_element_type=jnp.float32) * dv8_ref[...]


def _mid(agg1, hs1, dv16, dv8, b1, W2a, W2b):
    return pl.pallas_call(
        _mid_body,
        grid=(NCHUNK,),
        in_specs=[
            pl.BlockSpec((NC, BLK, F1), lambda i: (0, i, 0)),
            pl.BlockSpec((BLK, F1), lambda i: (i, 0)),
            pl.BlockSpec((BLK, F1), lambda i: (i, 0)),
            pl.BlockSpec((BLK, 8), lambda i: (i, 0)),
            pl.BlockSpec((1, F1), lambda i: (0, 0)),
            pl.BlockSpec((F1, 16), lambda i: (0, 0)),
            pl.BlockSpec((F1, 8), lambda i: (0, 0)),
        ],
        out_specs=[
            pl.BlockSpec((BLK, 16), lambda i: (i, 0)),
            pl.BlockSpec((BLK, 8), lambda i: (i, 0)),
        ],
        out_shape=[
            jax.ShapeDtypeStruct((N_PAD, 16), jnp.float32),
            jax.ShapeDtypeStruct((N_PAD, 8), jnp.float32),
        ],
    )(agg1, hs1, dv16, dv8, b1, W2a, W2b)


def _final_body(agg2a_ref, agg2b_ref, hs2a_ref, hs2b_ref, dv16_ref, dv8_ref,
                b2a_ref, b2b_ref, batch_ref, w3_ref, b3_ref, res_ref, acc_ref):
    i = pl.program_id(0)
    a = (agg2a_ref[0] + agg2a_ref[1] + hs2a_ref[...]) * dv16_ref[...] \
        + b2a_ref[...]
    b = (agg2b_ref[0] + agg2b_ref[1] + hs2b_ref[...]) * dv8_ref[...] \
        + b2b_ref[...]
    out2a = jnp.maximum(a, 0.0)
    out2b = jnp.maximum(b, 0.0)
    ones = jnp.ones((BLK, 1), jnp.float32)
    zeros = jnp.zeros((BLK, 7), jnp.float32)
    hext = jnp.concatenate([out2a, out2b, ones, zeros], axis=1)
    batch = batch_ref[...].reshape(BLK)
    mask = (lax.broadcasted_iota(jnp.int32, (G, BLK), 0)
            == batch[None, :]).astype(jnp.float32)

    @pl.when(i == 0)
    def _():
        acc_ref[...] = jnp.zeros((G, 32), jnp.float32)

    acc_ref[...] += jnp.dot(mask, hext, preferred_element_type=jnp.float32)

    @pl.when(i == NCHUNK - 1)
    def _():
        acc = acc_ref[...]
        pooled = acc[:, :24] / jnp.maximum(acc[:, 24:25], 1.0)
        res_ref[...] = jax.nn.sigmoid(
            jnp.dot(pooled, w3_ref[...], preferred_element_type=jnp.float32)
            + b3_ref[...])


def _final(agg2a, agg2b, hs2a, hs2b, dv16, dv8, b2a, b2b, batchp, W3, b3):
    return pl.pallas_call(
        _final_body,
        grid=(NCHUNK,),
        in_specs=[
            pl.BlockSpec((NC, BLK, 16), lambda i: (0, i, 0)),
            pl.BlockSpec((NC, BLK, 8), lambda i: (0, i, 0)),
            pl.BlockSpec((BLK, 16), lambda i: (i, 0)),
            pl.BlockSpec((BLK, 8), lambda i: (i, 0)),
            pl.BlockSpec((BLK, 16), lambda i: (i, 0)),
            pl.BlockSpec((BLK, 8), lambda i: (i, 0)),
            pl.BlockSpec((1, 16), lambda i: (0, 0)),
            pl.BlockSpec((1, 8), lambda i: (0, 0)),
            pl.BlockSpec((NSUB, CH), lambda i: (i, 0)),
            pl.BlockSpec((24, 1), lambda i: (0, 0)),
            pl.BlockSpec((1, 1), lambda i: (0, 0)),
        ],
        out_specs=pl.BlockSpec((G, 1), lambda i: (0, 0)),
        out_shape=jax.ShapeDtypeStruct((G, 1), jnp.float32),
        scratch_shapes=[pltpu.VMEM((G, 32), jnp.float32)],
    )(agg2a, agg2b, hs2a, hs2b, dv16, dv8, b2a, b2b, batchp, W3, b3)


def kernel(x, edge_index, batch, W1, b1, W2, b2, W3, b3):
    row = edge_index[0]
    col = edge_index[1]
    pad_e = E_PAD - E
    rowp = jnp.concatenate(
        [row, jnp.zeros((pad_e,), jnp.int32)]).reshape(ER, CH)
    colp = jnp.concatenate(
        [col, jnp.full((pad_e,), N, jnp.int32)]).reshape(ER, CH)
    xp = jnp.pad(x, ((0, N_PAD - N), (0, 0)))
    batchp = jnp.pad(batch, (0, N_PAD - N),
                     constant_values=G).reshape(NR, CH)

    ones_h = jnp.ones((CH,), jnp.float32)
    zeros_deg = jnp.zeros((RD,), jnp.float32)
    zeros16 = jnp.zeros((ZR, 16), jnp.float32)
    zeros8 = jnp.zeros((ZR, 8), jnp.float32)

    deg2 = _deg_pass(colp, ones_h, zeros_deg).reshape(NC, NR, CH)
    dinv = _dinv(deg2)
    dv16 = jnp.broadcast_to(dinv.reshape(N_PAD, 1), (N_PAD, 16))
    dv8 = jnp.broadcast_to(dinv.reshape(N_PAD, 1), (N_PAD, 8))
    hs1 = _prep1(xp, W1, dv16)
    agg1 = _edge_pass_16(hs1, rowp, colp, zeros16)
    hs2a, hs2b = _mid(agg1, hs1, dv16, dv8, b1.reshape(1, F1),
                      W2[:, :16], W2[:, 16:24])
    agg2a = _edge_pass_16(hs2a, rowp, colp, zeros16)
    agg2b = _edge_pass_8(hs2b, rowp, colp, zeros8)
    return _final(agg2a, agg2b, hs2a, hs2b, dv16, dv8,
                  b2[:16].reshape(1, 16), b2[16:24].reshape(1, 8),
                  batchp, W3, b3.reshape(1, 1))


# speedup vs baseline: 48.5045x; 1.7672x over previous
import functools

import jax
import jax.numpy as jnp
from jax import lax
from jax.experimental import pallas as pl
from jax.experimental.pallas import tpu as pltpu
from jax.experimental.pallas import tpu_sc as plsc

N = 100000
E = 3200000
G = 64
F1 = 16

NC, NS = 2, 16
NW = NC * NS
CH = 128
SJ = 16
BLK = 2048
NSUB = BLK // CH

N_PAD = 100352
NR = N_PAD // CH
NCHUNK = N_PAD // BLK
RD = N_PAD // NS
ZR = RD // 16
P = 8

ER = 25088
E_PAD = ER * CH
TR = ER // NW
OI = TR // SJ


def _edge_mesh():
    return plsc.VectorSubcoreMesh(core_axis_name="c", subcore_axis_name="s")


def _make_edge_pass(F):
    @functools.partial(
        pl.kernel,
        out_type=jax.ShapeDtypeStruct((NC, N_PAD, F), jnp.float32),
        mesh=_edge_mesh(),
        scratch_types=[
            pltpu.VMEM((SJ, CH), jnp.int32),
            pltpu.VMEM((SJ, CH), jnp.int32),
            pltpu.VMEM((P, CH, F), jnp.float32),
            pltpu.VMEM((ZR, F), jnp.float32),
            pltpu.VMEM_SHARED((N_PAD, F), jnp.float32),
            pltpu.SemaphoreType.DMA,
        ],
        compiler_params=pltpu.CompilerParams(use_tc_tiling_on_sc=False),
    )
    def edge_pass(hs, rows, cols, zc, out, rowb, colb, rv, zb, agg, sem):
        c = lax.axis_index("c")
        s = lax.axis_index("s")
        wid = s * NC + c

        pltpu.sync_copy(zc, zb)
        for z in range(RD // ZR):
            pltpu.sync_copy(zb, agg.at[pl.ds(s * RD + z * ZR, ZR)])
        plsc.subcore_barrier()

        def outer(i, carry):
            base = wid * TR + i * SJ
            pltpu.sync_copy(rows.at[pl.ds(base, SJ)], rowb)
            pltpu.sync_copy(cols.at[pl.ds(base, SJ)], colb)
            descs = [None] * SJ
            for j in range(P):
                descs[j] = pltpu.async_copy(hs.at[rowb.at[j]],
                                            rv.at[j % P], sem)
            for j in range(SJ):
                descs[j].wait()
                pltpu.sync_copy(rv.at[j % P], agg.at[colb.at[j]], add=True)
                jj = j + P
                if jj < SJ:
                    descs[jj] = pltpu.async_copy(hs.at[rowb.at[jj]],
                                                 rv.at[jj % P], sem)
            return carry

        lax.fori_loop(0, OI, outer, 0)
        plsc.subcore_barrier()

        for z in range(RD // ZR):
            sl = pl.ds(s * RD + z * ZR, ZR)
            pltpu.sync_copy(agg.at[sl], zb)
            pltpu.sync_copy(zb, out.at[c, sl])

    return edge_pass


_edge_pass_16 = _make_edge_pass(16)
_edge_pass_8 = _make_edge_pass(8)


@functools.partial(
    pl.kernel,
    out_type=jax.ShapeDtypeStruct((NC, N_PAD), jnp.float32),
    mesh=_edge_mesh(),
    scratch_types=[
        pltpu.VMEM((SJ, CH), jnp.int32),
        pltpu.VMEM((CH,), jnp.float32),
        pltpu.VMEM((RD,), jnp.float32),
        pltpu.VMEM_SHARED((N_PAD,), jnp.float32),
        pltpu.SemaphoreType.DMA,
    ],
    compiler_params=pltpu.CompilerParams(use_tc_tiling_on_sc=False),
)
def _deg_pass(cols, ones_h, zeros_h, out, colb, onesb, zb, deg, sem):
    c = lax.axis_index("c")
    s = lax.axis_index("s")
    wid = s * NC + c

    pltpu.sync_copy(ones_h, onesb)
    pltpu.sync_copy(zeros_h, zb)
    pltpu.sync_copy(zb, deg.at[pl.ds(s * RD, RD)])
    plsc.subcore_barrier()

    def outer(i, carry):
        base = wid * TR + i * SJ
        pltpu.sync_copy(cols.at[pl.ds(base, SJ)], colb)
        for j in range(SJ):
            pltpu.sync_copy(onesb, deg.at[colb.at[j]], add=True)
        return carry

    lax.fori_loop(0, OI, outer, 0)
    plsc.subcore_barrier()

    pltpu.sync_copy(deg.at[pl.ds(s * RD, RD)], zb)
    pltpu.sync_copy(zb, out.at[c, pl.ds(s * RD, RD)])


def _dinv_body(deg_ref, dinv_ref):
    deg = deg_ref[0] + deg_ref[1]
    dinv_ref[...] = jnp.where(deg > 0, lax.rsqrt(deg), 0.0)


def _dinv(deg2):
    return pl.pallas_call(
        _dinv_body,
        in_specs=[pl.BlockSpec((NC, NR, CH), lambda: (0, 0, 0))],
        out_specs=pl.BlockSpec((NR, CH), lambda: (0, 0)),
        out_shape=jax.ShapeDtypeStruct((NR, CH), jnp.float32),
    )(deg2)


def _prep1_body(x_ref, w1_ref, dv_ref, hs1_ref):
    h = jnp.dot(x_ref[...], w1_ref[...], preferred_element_type=jnp.float32)
    hs1_ref[...] = h * dv_ref[...]


def _prep1(xp, W1, dv16):
    return pl.pallas_call(
        _prep1_body,
        grid=(NCHUNK,),
        in_specs=[
            pl.BlockSpec((BLK, 8), lambda i: (i, 0)),
            pl.BlockSpec((8, F1), lambda i: (0, 0)),
            pl.BlockSpec((BLK, F1), lambda i: (i, 0)),
        ],
        out_specs=pl.BlockSpec((BLK, F1), lambda i: (i, 0)),
        out_shape=jax.ShapeDtypeStruct((N_PAD, F1), jnp.float32),
    )(xp, W1, dv16)


def _mid_body(agg1_ref, hs1_ref, dv16_ref, dv8_ref, b1_ref, w2a_ref, w2b_ref,
              hs2a_ref, hs2b_ref):
    dv = dv16_ref[...]
    a = agg1_ref[0] + agg1_ref[1] + hs1_ref[...]
    out1 = jnp.maximum(a * dv + b1_ref[...], 0.0)
    hs2a_ref[...] = jnp.dot(out1, w2a_ref[...],
                            preferred_element_type=jnp.float32) * dv
    hs2b_ref[...] = jnp.dot(out1, w2b_ref[...],
                            preferred_element_type=jnp.float32) * dv8_ref[...]


def _mid(agg1, hs1, dv16, dv8, b1, W2a, W2b):
    return pl.pallas_call(
        _mid_body,
        grid=(NCHUNK,),
        in_specs=[
            pl.BlockSpec((NC, BLK, F1), lambda i: (0, i, 0)),
            pl.BlockSpec((BLK, F1), lambda i: (i, 0)),
            pl.BlockSpec((BLK, F1), lambda i: (i, 0)),
            pl.BlockSpec((BLK, 8), lambda i: (i, 0)),
            pl.BlockSpec((1, F1), lambda i: (0, 0)),
            pl.BlockSpec((F1, 16), lambda i: (0, 0)),
            pl.BlockSpec((F1, 8), lambda i: (0, 0)),
        ],
        out_specs=[
            pl.BlockSpec((BLK, 16), lambda i: (i, 0)),
            pl.BlockSpec((BLK, 8), lambda i: (i, 0)),
        ],
        out_shape=[
            jax.ShapeDtypeStruct((N_PAD, 16), jnp.float32),
            jax.ShapeDtypeStruct((N_PAD, 8), jnp.float32),
        ],
    )(agg1, hs1, dv16, dv8, b1, W2a, W2b)


def _final_body(agg2a_ref, agg2b_ref, hs2a_ref, hs2b_ref, dv16_ref, dv8_ref,
                b2a_ref, b2b_ref, batch_ref, w3_ref, b3_ref, res_ref, acc_ref):
    i = pl.program_id(0)
    a = (agg2a_ref[0] + agg2a_ref[1] + hs2a_ref[...]) * dv16_ref[...] \
        + b2a_ref[...]
    b = (agg2b_ref[0] + agg2b_ref[1] + hs2b_ref[...]) * dv8_ref[...] \
        + b2b_ref[...]
    out2a = jnp.maximum(a, 0.0)
    out2b = jnp.maximum(b, 0.0)
    ones = jnp.ones((BLK, 1), jnp.float32)
    zeros = jnp.zeros((BLK, 7), jnp.float32)
    hext = jnp.concatenate([out2a, out2b, ones, zeros], axis=1)
    batch = batch_ref[...].reshape(BLK)
    mask = (lax.broadcasted_iota(jnp.int32, (G, BLK), 0)
            == batch[None, :]).astype(jnp.float32)

    @pl.when(i == 0)
    def _():
        acc_ref[...] = jnp.zeros((G, 32), jnp.float32)

    acc_ref[...] += jnp.dot(mask, hext, preferred_element_type=jnp.float32)

    @pl.when(i == NCHUNK - 1)
    def _():
        acc = acc_ref[...]
        pooled = acc[:, :24] / jnp.maximum(acc[:, 24:25], 1.0)
        res_ref[...] = jax.nn.sigmoid(
            jnp.dot(pooled, w3_ref[...], preferred_element_type=jnp.float32)
            + b3_ref[...])


def _final(agg2a, agg2b, hs2a, hs2b, dv16, dv8, b2a, b2b, batchp, W3, b3):
    return pl.pallas_call(
        _final_body,
        grid=(NCHUNK,),
        in_specs=[
            pl.BlockSpec((NC, BLK, 16), lambda i: (0, i, 0)),
            pl.BlockSpec((NC, BLK, 8), lambda i: (0, i, 0)),
            pl.BlockSpec((BLK, 16), lambda i: (i, 0)),
            pl.BlockSpec((BLK, 8), lambda i: (i, 0)),
            pl.BlockSpec((BLK, 16), lambda i: (i, 0)),
            pl.BlockSpec((BLK, 8), lambda i: (i, 0)),
            pl.BlockSpec((1, 16), lambda i: (0, 0)),
            pl.BlockSpec((1, 8), lambda i: (0, 0)),
            pl.BlockSpec((NSUB, CH), lambda i: (i, 0)),
            pl.BlockSpec((24, 1), lambda i: (0, 0)),
            pl.BlockSpec((1, 1), lambda i: (0, 0)),
        ],
        out_specs=pl.BlockSpec((G, 1), lambda i: (0, 0)),
        out_shape=jax.ShapeDtypeStruct((G, 1), jnp.float32),
        scratch_shapes=[pltpu.VMEM((G, 32), jnp.float32)],
    )(agg2a, agg2b, hs2a, hs2b, dv16, dv8, b2a, b2b, batchp, W3, b3)


def kernel(x, edge_index, batch, W1, b1, W2, b2, W3, b3):
    row = edge_index[0]
    col = edge_index[1]
    pad_e = E_PAD - E
    rowp = jnp.concatenate(
        [row, jnp.zeros((pad_e,), jnp.int32)]).reshape(ER, CH)
    colp = jnp.concatenate(
        [col, jnp.full((pad_e,), N, jnp.int32)]).reshape(ER, CH)
    xp = jnp.pad(x, ((0, N_PAD - N), (0, 0)))
    batchp = jnp.pad(batch, (0, N_PAD - N),
                     constant_values=G).reshape(NR, CH)

    ones_h = jnp.ones((CH,), jnp.float32)
    zeros_deg = jnp.zeros((RD,), jnp.float32)
    zeros16 = jnp.zeros((ZR, 16), jnp.float32)
    zeros8 = jnp.zeros((ZR, 8), jnp.float32)

    deg2 = _deg_pass(colp, ones_h, zeros_deg).reshape(NC, NR, CH)
    dinv = _dinv(deg2)
    dv16 = jnp.broadcast_to(dinv.reshape(N_PAD, 1), (N_PAD, 16))
    dv8 = jnp.broadcast_to(dinv.reshape(N_PAD, 1), (N_PAD, 8))
    hs1 = _prep1(xp, W1, dv16)
    agg1 = _edge_pass_16(hs1, rowp, colp, zeros16)
    hs2a, hs2b = _mid(agg1, hs1, dv16, dv8, b1.reshape(1, F1),
                      W2[:, :16], W2[:, 16:24])
    agg2a = _edge_pass_16(hs2a, rowp, colp, zeros16)
    agg2b = _edge_pass_8(hs2b, rowp, colp, zeros8)
    return _final(agg2a, agg2b, hs2a, hs2b, dv16, dv8,
                  b2[:16].reshape(1, 16), b2[16:24].reshape(1, 8),
                  batchp, W3, b3.reshape(1, 1))
